# baseline (speedup 1.0000x reference)
import jax
import jax.numpy as jnp
from jax.experimental import pallas as pl
from jax.experimental.pallas import tpu as pltpu

B, N, F = 4, 2048, 128
H = 256
C = 128
OUT = 36
BN_EPS = 1e-3

T = 2048
NT = N // T


def _body(eps_ref, a_ref, x_ref, W1_ref, b1_ref, g1_ref, be1_ref, m1_ref,
          v1_ref, Wf_ref, bf_ref, g2_ref, be2_ref, m2_ref, v2_ref, Wd_ref,
          bd_ref, out_ref, acc_ref):
    i = pl.program_id(1)

    a_tile = a_ref[0]
    x_full = x_ref[0]
    agg = jnp.dot(a_tile, x_full, preferred_element_type=jnp.float32)

    x_tile = x_ref[0, pl.ds(i * T, T), :]
    h = (1.0 + eps_ref[0, 0]) * x_tile + agg

    h = jnp.dot(h, W1_ref[...], preferred_element_type=jnp.float32) + b1_ref[...]
    s1 = g1_ref[...] * jax.lax.rsqrt(v1_ref[...] + BN_EPS)
    h = jnp.maximum(h * s1 + (be1_ref[...] - m1_ref[...] * s1), 0.0)

    h = jnp.maximum(
        jnp.dot(h, Wf_ref[...], preferred_element_type=jnp.float32) + bf_ref[...],
        0.0)
    s2 = g2_ref[...] * jax.lax.rsqrt(v2_ref[...] + BN_EPS)
    h = h * s2 + (be2_ref[...] - m2_ref[...] * s2)

    tile_max = jnp.max(h, axis=0, keepdims=True)

    @pl.when(i == 0)
    def _():
        acc_ref[...] = jnp.full((8, C), -jnp.inf, dtype=jnp.float32)

    acc_ref[0:1, :] = jnp.maximum(acc_ref[0:1, :], tile_max)

    @pl.when(i == NT - 1)
    def _():
        p = acc_ref[0:1, :]
        out_ref[...] = (jnp.dot(p, Wd_ref[...],
                                preferred_element_type=jnp.float32)
                        + bd_ref[...]).reshape(1, 1, OUT)


@jax.jit
def kernel(x, a, eps, W1, b1, g1, be1, m1, v1, Wf, bf, g2, be2, m2, v2, Wd, bd):
    eps2 = eps.reshape(1, 1)
    vecs = [v.reshape(1, -1) for v in (b1, g1, be1, m1, v1, bf, g2, be2, m2, v2, bd)]
    b1r, g1r, be1r, m1r, v1r, bfr, g2r, be2r, m2r, v2r, bdr = vecs

    full = lambda shape: pl.BlockSpec(shape, lambda b, i: (0,) * len(shape))
    out = pl.pallas_call(
        _body,
        grid=(B, NT),
        in_specs=[
            pl.BlockSpec(memory_space=pltpu.SMEM),
            pl.BlockSpec((1, T, N), lambda b, i: (b, i, 0)),
            pl.BlockSpec((1, N, F), lambda b, i: (b, 0, 0)),
            full((F, H)),
            full((1, H)), full((1, H)), full((1, H)), full((1, H)), full((1, H)),
            full((H, C)),
            full((1, C)), full((1, C)), full((1, C)), full((1, C)), full((1, C)),
            full((C, OUT)),
            full((1, OUT)),
        ],
        out_specs=pl.BlockSpec((1, 1, OUT), lambda b, i: (b, 0, 0)),
        out_shape=jax.ShapeDtypeStruct((B, 1, OUT), jnp.float32),
        scratch_shapes=[pltpu.VMEM((8, C), jnp.float32)],
    )(eps2, a, x, W1, b1r, g1r, be1r, m1r, v1r, Wf, bfr, g2r, be2r, m2r,
      v2r, Wd, bdr)
    return out.reshape(B, OUT)


# speedup vs baseline: 1.0261x; 1.0261x over previous
import jax
import jax.numpy as jnp
from jax.experimental import pallas as pl
from jax.experimental.pallas import tpu as pltpu

B, N, F = 4, 2048, 128
H = 256
C = 128
OUT = 36
BN_EPS = 1e-3

T = 1024
NT = N // T


def _body(eps_ref, a_ref, x_ref, W1_ref, b1_ref, g1_ref, be1_ref, m1_ref,
          v1_ref, Wf_ref, bf_ref, g2_ref, be2_ref, m2_ref, v2_ref, Wd_ref,
          bd_ref, out_ref, acc_ref):
    i = pl.program_id(1)

    a_tile = a_ref[0].astype(jnp.bfloat16)
    x_full = x_ref[0].astype(jnp.bfloat16)
    agg = jnp.dot(a_tile, x_full, preferred_element_type=jnp.float32)

    x_tile = x_ref[0, pl.ds(i * T, T), :]
    h = (1.0 + eps_ref[0, 0]) * x_tile + agg

    h = jnp.dot(h, W1_ref[...], preferred_element_type=jnp.float32) + b1_ref[...]
    s1 = g1_ref[...] * jax.lax.rsqrt(v1_ref[...] + BN_EPS)
    h = jnp.maximum(h * s1 + (be1_ref[...] - m1_ref[...] * s1), 0.0)

    h = jnp.maximum(
        jnp.dot(h, Wf_ref[...], preferred_element_type=jnp.float32) + bf_ref[...],
        0.0)
    s2 = g2_ref[...] * jax.lax.rsqrt(v2_ref[...] + BN_EPS)
    h = h * s2 + (be2_ref[...] - m2_ref[...] * s2)

    tile_max = jnp.max(h, axis=0, keepdims=True)

    @pl.when(i == 0)
    def _():
        acc_ref[...] = jnp.full((8, C), -jnp.inf, dtype=jnp.float32)

    acc_ref[0:1, :] = jnp.maximum(acc_ref[0:1, :], tile_max)

    @pl.when(i == NT - 1)
    def _():
        p = acc_ref[0:1, :]
        out_ref[...] = (jnp.dot(p, Wd_ref[...],
                                preferred_element_type=jnp.float32)
                        + bd_ref[...]).reshape(1, 1, OUT)


@jax.jit
def kernel(x, a, eps, W1, b1, g1, be1, m1, v1, Wf, bf, g2, be2, m2, v2, Wd, bd):
    eps2 = eps.reshape(1, 1)
    vecs = [v.reshape(1, -1) for v in (b1, g1, be1, m1, v1, bf, g2, be2, m2, v2, bd)]
    b1r, g1r, be1r, m1r, v1r, bfr, g2r, be2r, m2r, v2r, bdr = vecs

    full = lambda shape: pl.BlockSpec(shape, lambda b, i: (0,) * len(shape))
    out = pl.pallas_call(
        _body,
        grid=(B, NT),
        in_specs=[
            pl.BlockSpec(memory_space=pltpu.SMEM),
            pl.BlockSpec((1, T, N), lambda b, i: (b, i, 0)),
            pl.BlockSpec((1, N, F), lambda b, i: (b, 0, 0)),
            full((F, H)),
            full((1, H)), full((1, H)), full((1, H)), full((1, H)), full((1, H)),
            full((H, C)),
            full((1, C)), full((1, C)), full((1, C)), full((1, C)), full((1, C)),
            full((C, OUT)),
            full((1, OUT)),
        ],
        out_specs=pl.BlockSpec((1, 1, OUT), lambda b, i: (b, 0, 0)),
        out_shape=jax.ShapeDtypeStruct((B, 1, OUT), jnp.float32),
        scratch_shapes=[pltpu.VMEM((8, C), jnp.float32)],
    )(eps2, a, x, W1, b1r, g1r, be1r, m1r, v1r, Wf, bfr, g2r, be2r, m2r,
      v2r, Wd, bdr)
    return out.reshape(B, OUT)


# speedup vs baseline: 1.0461x; 1.0195x over previous
import jax
import jax.numpy as jnp
from jax.experimental import pallas as pl
from jax.experimental.pallas import tpu as pltpu

B, N, F = 4, 2048, 128
H = 256
C = 128
OUT = 36
BN_EPS = 1e-3

T = 1024
NT = N // T


def _body(eps_ref, a_ref, x_ref, W1_ref, b1_ref, g1_ref, be1_ref, m1_ref,
          v1_ref, Wf_ref, bf_ref, g2_ref, be2_ref, m2_ref, v2_ref, Wd_ref,
          bd_ref, out_ref, acc_ref):
    i = pl.program_id(1)

    b = pl.program_id(0)
    a_tile = a_ref[0]
    x_full = x_ref[b]
    agg = jnp.dot(a_tile, x_full, preferred_element_type=jnp.float32)

    x_tile = x_ref[b, pl.ds(i * T, T), :]
    h = (1.0 + eps_ref[0, 0]) * x_tile + agg

    h = jnp.dot(h, W1_ref[...], preferred_element_type=jnp.float32) + b1_ref[...]
    s1 = g1_ref[...] * jax.lax.rsqrt(v1_ref[...] + BN_EPS)
    h = jnp.maximum(h * s1 + (be1_ref[...] - m1_ref[...] * s1), 0.0)

    h = jnp.maximum(
        jnp.dot(h, Wf_ref[...], preferred_element_type=jnp.float32) + bf_ref[...],
        0.0)
    s2 = g2_ref[...] * jax.lax.rsqrt(v2_ref[...] + BN_EPS)
    h = h * s2 + (be2_ref[...] - m2_ref[...] * s2)

    tile_max = jnp.max(h, axis=0, keepdims=True)

    @pl.when(i == 0)
    def _():
        acc_ref[...] = jnp.full((8, C), -jnp.inf, dtype=jnp.float32)

    acc_ref[0:1, :] = jnp.maximum(acc_ref[0:1, :], tile_max)

    @pl.when(i == NT - 1)
    def _():
        p = acc_ref[0:1, :]
        out_ref[...] = (jnp.dot(p, Wd_ref[...],
                                preferred_element_type=jnp.float32)
                        + bd_ref[...]).reshape(1, 1, OUT)


@jax.jit
def kernel(x, a, eps, W1, b1, g1, be1, m1, v1, Wf, bf, g2, be2, m2, v2, Wd, bd):
    eps2 = eps.reshape(1, 1)
    vecs = [v.reshape(1, -1) for v in (b1, g1, be1, m1, v1, bf, g2, be2, m2, v2, bd)]
    b1r, g1r, be1r, m1r, v1r, bfr, g2r, be2r, m2r, v2r, bdr = vecs

    full = lambda shape: pl.BlockSpec(shape, lambda b, i: (0,) * len(shape))
    out = pl.pallas_call(
        _body,
        grid=(B, NT),
        in_specs=[
            pl.BlockSpec(memory_space=pltpu.SMEM),
            pl.BlockSpec((1, T, N), lambda b, i: (b, i, 0)),
            pl.BlockSpec((B, N, F), lambda b, i: (0, 0, 0)),
            full((F, H)),
            full((1, H)), full((1, H)), full((1, H)), full((1, H)), full((1, H)),
            full((H, C)),
            full((1, C)), full((1, C)), full((1, C)), full((1, C)), full((1, C)),
            full((C, OUT)),
            full((1, OUT)),
        ],
        out_specs=pl.BlockSpec((1, 1, OUT), lambda b, i: (b, 0, 0)),
        out_shape=jax.ShapeDtypeStruct((B, 1, OUT), jnp.float32),
        scratch_shapes=[pltpu.VMEM((8, C), jnp.float32)],
    )(eps2, a, x, W1, b1r, g1r, be1r, m1r, v1r, Wf, bfr, g2r, be2r, m2r,
      v2r, Wd, bdr)
    return out.reshape(B, OUT)
